# baseline (speedup 1.0000x reference)
import functools

import jax
import jax.numpy as jnp
from jax import lax
from jax.experimental import pallas as pl
from jax.experimental.pallas import tpu as pltpu
from jax.experimental.pallas import tpu_sc as plsc

_B, _S, _D = 4, 8192, 1024
_NC, _NS = 2, 16
_NW = _NC * _NS
_RPW = _S // _NW
_CH = 32
_NCHUNK = _RPW // _CH


def _sc_body(w_hbm, o_hbm, buf0, buf1, sem_r0, sem_r1, sem_w):
    c = lax.axis_index("c")
    s = lax.axis_index("s")
    wid = s * _NC + c
    base = wid * _RPW
    bufs = (buf0, buf1)
    sems = (sem_r0, sem_r1)

    def start_read(i):
        return pltpu.async_copy(
            w_hbm.at[pl.ds(base + i * _CH, _CH)], bufs[i % 2], sems[i % 2])

    reads = {0: start_read(0)}
    writes_prev = []
    for i in range(_NCHUNK):
        for h in writes_prev:
            h.wait()
        if i + 1 < _NCHUNK:
            reads[i + 1] = start_read(i + 1)
        reads[i].wait()
        writes_prev = [
            pltpu.async_copy(
                bufs[i % 2], o_hbm.at[b, pl.ds(base + i * _CH, _CH)], sem_w)
            for b in range(_B)
        ]
    for h in writes_prev:
        h.wait()


@functools.partial(
    pl.kernel,
    out_type=jax.ShapeDtypeStruct((_B, _S, _D), jnp.float32),
    mesh=plsc.VectorSubcoreMesh(core_axis_name="c", subcore_axis_name="s"),
    scratch_types=[
        pltpu.VMEM((_CH, _D), jnp.float32),
        pltpu.VMEM((_CH, _D), jnp.float32),
        pltpu.SemaphoreType.DMA,
        pltpu.SemaphoreType.DMA,
        pltpu.SemaphoreType.DMA,
    ],
)
def _sc_broadcast_copy(w_hbm, o_hbm, buf0, buf1, sem_r0, sem_r1, sem_w):
    _sc_body(w_hbm, o_hbm, buf0, buf1, sem_r0, sem_r1, sem_w)


def kernel(x, weight):
    return _sc_broadcast_copy(weight)
